# baseline (speedup 1.0000x reference)
import functools

import jax
import jax.numpy as jnp
from jax import lax
from jax.experimental import pallas as pl
from jax.experimental.pallas import tpu as pltpu
from jax.experimental.pallas import tpu_sc as plsc

N = 10000
NPAD = 10240
E = 160000
F = 256
HN = NPAD // 2
K = 2
T = 2
NC = 2
NS = 16
CW = 32
NCH = E // CW
PADC = 2 * NS
MAXC = 320
NCHP = NCH + PADC + MAXC
EP = NCHP * CW
EPAD = 163840
DCH = EPAD // (NC * NS * 128)
RPT = HN // NS
BN = 256


def _mesh():
    return plsc.VectorSubcoreMesh(core_axis_name="c", subcore_axis_name="s")


def _deg_body(dst_hbm, out_hbm, dstv, onesv, zv, acc):
    c = lax.axis_index("c")
    s = lax.axis_index("s")
    w = s * NC + c

    for v8 in range(8):
        onesv[pl.ds(v8 * 16, 16)] = jnp.ones((16,), jnp.float32)

    @pl.loop(0, (NPAD // NS) // 16)
    def _(r):
        zv[pl.ds(r * 16, 16)] = jnp.zeros((16,), jnp.float32)

    pltpu.sync_copy(dst_hbm.at[w], dstv)
    pltpu.sync_copy(zv, acc.at[pl.ds(s * (NPAD // NS), NPAD // NS)])
    plsc.subcore_barrier()

    @pl.loop(0, DCH)
    def _(j):
        pltpu.sync_copy(onesv, acc.at[dstv.at[j]], add=True)

    plsc.subcore_barrier()
    pltpu.sync_copy(acc.at[pl.ds(s * (NPAD // NS), NPAD // NS)],
                    out_hbm.at[c, pl.ds(s * (NPAD // NS), NPAD // NS)])


def _deg_call(dst_d):
    fn = pl.kernel(
        _deg_body,
        out_type=jax.ShapeDtypeStruct((NC, NPAD), jnp.float32),
        mesh=_mesh(),
        compiler_params=pltpu.CompilerParams(use_tc_tiling_on_sc=False),
        scratch_types=[
            pltpu.VMEM((DCH, 128), jnp.int32),
            pltpu.VMEM((128,), jnp.float32),
            pltpu.VMEM((NPAD // NS,), jnp.float32),
            pltpu.VMEM_SHARED((NPAD,), jnp.float32),
        ],
    )
    return fn(dst_d)


def _prop_body(h_hbm, src_hbm, dst_hbm, cnt_hbm, out_hbm,
               src_v, dst_v, dbuf, zbuf, cntv, acc, sem0, sem1):
    c = lax.axis_index("c")
    s = lax.axis_index("s")

    pltpu.sync_copy(cnt_hbm.at[c], cntv)
    vec = cntv[...]
    bc = vec[0]
    cpt = vec[1]
    lo = bc + s * cpt

    pltpu.sync_copy(src_hbm.at[c, pl.ds(lo, MAXC)], src_v)
    pltpu.sync_copy(dst_hbm.at[c, pl.ds(lo, MAXC)], dst_v)

    @pl.loop(0, zbuf.shape[0])
    def _(r):
        for v in range(F // 16):
            zbuf[r, pl.ds(v * 16, 16)] = jnp.zeros((16,), jnp.float32)

    m = cpt // 2
    for k in range(K):
        if k:
            @pl.loop(0, MAXC)
            def _(r):
                for v in range(CW // 16):
                    sl = pl.ds(v * 16, 16)
                    src_v[r, sl] = src_v[r, sl] + NPAD

        for z in range(RPT // zbuf.shape[0]):
            pltpu.sync_copy(
                zbuf, acc.at[pl.ds(s * RPT + z * zbuf.shape[0],
                                   zbuf.shape[0])])
        plsc.subcore_barrier()

        pltpu.async_copy(h_hbm.at[src_v.at[0]], dbuf.at[0], sem0)

        @pl.loop(0, m - 1)
        def _(jj):
            j = jj * 2
            pltpu.async_copy(h_hbm.at[src_v.at[j + 1]], dbuf.at[1], sem1)
            pltpu.make_async_copy(h_hbm.at[src_v.at[j]], dbuf.at[0], sem0).wait()
            pltpu.sync_copy(dbuf.at[0], acc.at[dst_v.at[j]], add=True)
            pltpu.async_copy(h_hbm.at[src_v.at[j + 2]], dbuf.at[0], sem0)
            pltpu.make_async_copy(h_hbm.at[src_v.at[j + 1]], dbuf.at[1], sem1).wait()
            pltpu.sync_copy(dbuf.at[1], acc.at[dst_v.at[j + 1]], add=True)

        last = m * 2 - 1
        pltpu.async_copy(h_hbm.at[src_v.at[last]], dbuf.at[1], sem1)
        pltpu.make_async_copy(h_hbm.at[src_v.at[last - 1]], dbuf.at[0], sem0).wait()
        pltpu.sync_copy(dbuf.at[0], acc.at[dst_v.at[last - 1]], add=True)
        pltpu.make_async_copy(h_hbm.at[src_v.at[last]], dbuf.at[1], sem1).wait()
        pltpu.sync_copy(dbuf.at[1], acc.at[dst_v.at[last]], add=True)

        plsc.subcore_barrier()
        pltpu.sync_copy(acc.at[pl.ds(s * RPT, RPT)],
                        out_hbm.at[k, c, pl.ds(s * RPT, RPT)])


def _prop_call(hs_flat, src_p, dst_p, cnt):
    fn = pl.kernel(
        _prop_body,
        out_type=jax.ShapeDtypeStruct((K, NC, HN, F), jnp.float32),
        mesh=_mesh(),
        compiler_params=pltpu.CompilerParams(use_tc_tiling_on_sc=False),
        scratch_types=[
            pltpu.VMEM((MAXC, CW), jnp.int32),
            pltpu.VMEM((MAXC, CW), jnp.int32),
            pltpu.VMEM((2, CW, F), jnp.float32),
            pltpu.VMEM((16, F), jnp.float32),
            pltpu.VMEM((16,), jnp.int32),
            pltpu.VMEM_SHARED((HN, F), jnp.float32),
            pltpu.SemaphoreType.DMA,
            pltpu.SemaphoreType.DMA,
        ],
    )
    return fn(hs_flat, src_p, dst_p, cnt)


def _rowmask(h):
    rows = (lax.broadcasted_iota(jnp.int32, (BN, 1), 0)
            + pl.program_id(0) * BN)
    return jnp.where(rows < N, h, 0.0)


def _tc0_body(x_ref, deg_ref, iw_ref, rw_ref, b_ref, hs_ref, r_ref, dis_ref):
    xb = x_ref[...]
    deg = deg_ref[0] + deg_ref[1]
    dis = jnp.where(deg > 0, lax.rsqrt(deg), 0.0)
    dis_ref[...] = dis
    for k in range(K):
        hs_ref[k] = jnp.dot(xb, iw_ref[k], preferred_element_type=jnp.float32) * dis
    for t in range(T):
        for k in range(K):
            r_ref[t, k] = (jnp.dot(xb, rw_ref[t, k],
                                   preferred_element_type=jnp.float32)
                           + b_ref[t, k])


def _tc0_call(xpad, deg2c, init_w, root_w, b):
    grid = (NPAD // BN,)
    return pl.pallas_call(
        _tc0_body,
        grid=grid,
        in_specs=[
            pl.BlockSpec((BN, F), lambda i: (i, 0)),
            pl.BlockSpec((NC, BN, 1), lambda i: (0, i, 0)),
            pl.BlockSpec((K, F, F), lambda i: (0, 0, 0)),
            pl.BlockSpec((T, K, F, F), lambda i: (0, 0, 0, 0)),
            pl.BlockSpec((T, K, 1, F), lambda i: (0, 0, 0, 0)),
        ],
        out_specs=[
            pl.BlockSpec((K, BN, F), lambda i: (0, i, 0)),
            pl.BlockSpec((T, K, BN, F), lambda i: (0, 0, i, 0)),
            pl.BlockSpec((BN, 1), lambda i: (i, 0)),
        ],
        out_shape=[
            jax.ShapeDtypeStruct((K, NPAD, F), jnp.float32),
            jax.ShapeDtypeStruct((T, K, NPAD, F), jnp.float32),
            jax.ShapeDtypeStruct((NPAD, 1), jnp.float32),
        ],
    )(xpad, deg2c, init_w, root_w, b)


def _tc1_body(p_ref, r_ref, dis_ref, w_ref, hs_ref):
    dis = dis_ref[...]
    for k in range(K):
        o = jnp.maximum(p_ref[k] * dis + r_ref[0, k], 0.0)
        h = jnp.dot(o, w_ref[0, k], preferred_element_type=jnp.float32) * dis
        hs_ref[k] = _rowmask(h)


def _tc1_call(p, r, dis, w):
    grid = (NPAD // BN,)
    return pl.pallas_call(
        _tc1_body,
        grid=grid,
        in_specs=[
            pl.BlockSpec((K, BN, F), lambda i: (0, i, 0)),
            pl.BlockSpec((T, K, BN, F), lambda i: (0, 0, i, 0)),
            pl.BlockSpec((BN, 1), lambda i: (i, 0)),
            pl.BlockSpec((T - 1, K, F, F), lambda i: (0, 0, 0, 0)),
        ],
        out_specs=[
            pl.BlockSpec((K, BN, F), lambda i: (0, i, 0)),
        ],
        out_shape=[
            jax.ShapeDtypeStruct((K, NPAD, F), jnp.float32),
        ],
    )(p, r, dis, w)[0]


def _tc2_body(p_ref, r_ref, dis_ref, iw_ref, rw_ref, b_ref, hs_ref, rn_ref):
    dis = dis_ref[...]
    outs = []
    for k in range(K):
        outs.append(jnp.maximum(p_ref[k] * dis + r_ref[1, k], 0.0))
    xs = 0.5 * (outs[0] + outs[1])
    for k in range(K):
        h = jnp.dot(xs, iw_ref[k], preferred_element_type=jnp.float32) * dis
        hs_ref[k] = _rowmask(h)
    for t in range(T):
        for k in range(K):
            rn_ref[t, k] = (jnp.dot(xs, rw_ref[t, k],
                                    preferred_element_type=jnp.float32)
                            + b_ref[t, k])


def _tc2_call(p, r, dis, init_w, root_w, b):
    grid = (NPAD // BN,)
    return pl.pallas_call(
        _tc2_body,
        grid=grid,
        in_specs=[
            pl.BlockSpec((K, BN, F), lambda i: (0, i, 0)),
            pl.BlockSpec((T, K, BN, F), lambda i: (0, 0, i, 0)),
            pl.BlockSpec((BN, 1), lambda i: (i, 0)),
            pl.BlockSpec((K, F, F), lambda i: (0, 0, 0)),
            pl.BlockSpec((T, K, F, F), lambda i: (0, 0, 0, 0)),
            pl.BlockSpec((T, K, 1, F), lambda i: (0, 0, 0, 0)),
        ],
        out_specs=[
            pl.BlockSpec((K, BN, F), lambda i: (0, i, 0)),
            pl.BlockSpec((T, K, BN, F), lambda i: (0, 0, i, 0)),
        ],
        out_shape=[
            jax.ShapeDtypeStruct((K, NPAD, F), jnp.float32),
            jax.ShapeDtypeStruct((T, K, NPAD, F), jnp.float32),
        ],
    )(p, r, dis, init_w, root_w, b)


def _tcf_body(p_ref, r_ref, dis_ref, out_ref):
    dis = dis_ref[...]
    acc = None
    for k in range(K):
        o = jnp.maximum(p_ref[k] * dis + r_ref[1, k], 0.0)
        acc = o if acc is None else acc + o
    out_ref[...] = 0.5 * acc


def _tcf_call(p, r, dis):
    grid = (NPAD // BN,)
    return pl.pallas_call(
        _tcf_body,
        grid=grid,
        in_specs=[
            pl.BlockSpec((K, BN, F), lambda i: (0, i, 0)),
            pl.BlockSpec((T, K, BN, F), lambda i: (0, 0, i, 0)),
            pl.BlockSpec((BN, 1), lambda i: (i, 0)),
        ],
        out_specs=[pl.BlockSpec((BN, F), lambda i: (i, 0))],
        out_shape=[jax.ShapeDtypeStruct((NPAD, F), jnp.float32)],
    )(p, r, dis)[0]


def _ceil_mult(x, mlt):
    return ((x + mlt - 1) // mlt) * mlt


def kernel(x, edge_index, init_w0, w0, root_w0, b0, init_w1, w1, root_w1, b1,
           init_w2, w2, root_w2, b2):
    src = edge_index[0]
    dst = edge_index[1]

    key = (dst >= HN).astype(jnp.int32)
    count0 = E - jnp.sum(key)
    cum0 = jnp.cumsum(1 - key)
    cum1 = jnp.cumsum(key)
    newpos = jnp.where(key == 0, cum0 - 1, count0 + cum1 - 1)
    pos = jnp.arange(EP, dtype=jnp.int32)
    junk_src = N + pos % (NPAD - N)
    srcS = junk_src.at[newpos].set(src)
    dstS = jnp.zeros((EP,), jnp.int32).at[newpos].set(dst)
    valid0 = pos < count0
    valid1 = (pos >= count0) & (pos < E)
    src0 = jnp.where(valid0, srcS, junk_src)
    dst0 = jnp.where(valid0, dstS, 0)
    src1 = jnp.where(valid1, srcS, junk_src)
    dst1 = jnp.where(valid1, dstS - HN, 0)
    src_p = jnp.stack([src0, src1]).reshape(NC, NCHP, CW)
    dst_p = jnp.stack([dst0, dst1]).reshape(NC, NCHP, CW)

    ch0 = (count0 + CW - 1) // CW
    cpt0 = jnp.maximum(_ceil_mult(ch0, PADC) // NS, 2)
    bc1 = count0 // CW
    cpt1 = jnp.maximum(_ceil_mult(NCH - bc1, PADC) // NS, 2)
    cnt = (jnp.zeros((NC, 16), jnp.int32)
           .at[0, 1].set(cpt0)
           .at[1, 0].set(bc1)
           .at[1, 1].set(cpt1))

    padd = N + jnp.arange(EPAD - E, dtype=jnp.int32) % (NPAD - N)
    dst_d = jnp.concatenate([dst, padd]).reshape(NC * NS, DCH, 128)
    xpad = jnp.pad(x, ((0, NPAD - N), (0, 0)))

    deg2 = _deg_call(dst_d)
    deg2c = deg2[:, :, None]

    layers = [(init_w0, w0, root_w0, b0),
              (init_w1, w1, root_w1, b1),
              (init_w2, w2, root_w2, b2)]

    hs, r, dis = _tc0_call(xpad, deg2c, layers[0][0], layers[0][2], layers[0][3])
    for li in range(3):
        p0 = _prop_call(hs.reshape(K * NPAD, F), src_p, dst_p, cnt)
        hs1 = _tc1_call(p0.reshape(K, NPAD, F), r, dis, layers[li][1])
        p1 = _prop_call(hs1.reshape(K * NPAD, F), src_p, dst_p, cnt)
        if li < 2:
            hs, r = _tc2_call(p1.reshape(K, NPAD, F), r, dis, layers[li + 1][0],
                              layers[li + 1][2], layers[li + 1][3])
        else:
            out = _tcf_call(p1.reshape(K, NPAD, F), r, dis)
    return out[:N]


# speedup vs baseline: 1.7131x; 1.7131x over previous
import functools

import jax
import jax.numpy as jnp
from jax import lax
from jax.experimental import pallas as pl
from jax.experimental.pallas import tpu as pltpu
from jax.experimental.pallas import tpu_sc as plsc

N = 10000
NPAD = 10240
E = 160000
F = 256
FH = 128
NH = 2
K = 2
T = 2
NC = 2
NS = 16
CW = 80
CH = 64
EPAD = NC * NS * CH * CW
DCH = EPAD // (NC * NS * 128)
RPT = NPAD // NS
BN = 256


def _mesh():
    return plsc.VectorSubcoreMesh(core_axis_name="c", subcore_axis_name="s")


def _deg_body(dst_hbm, out_hbm, dstv, onesv, zv, acc):
    c = lax.axis_index("c")
    s = lax.axis_index("s")
    w = s * NC + c

    for v8 in range(8):
        onesv[pl.ds(v8 * 16, 16)] = jnp.ones((16,), jnp.float32)

    @pl.loop(0, RPT // 16)
    def _(r):
        zv[pl.ds(r * 16, 16)] = jnp.zeros((16,), jnp.float32)

    pltpu.sync_copy(dst_hbm.at[w], dstv)
    pltpu.sync_copy(zv, acc.at[pl.ds(s * RPT, RPT)])
    plsc.subcore_barrier()

    @pl.loop(0, DCH)
    def _(j):
        pltpu.sync_copy(onesv, acc.at[dstv.at[j]], add=True)

    plsc.subcore_barrier()
    pltpu.sync_copy(acc.at[pl.ds(s * RPT, RPT)], out_hbm.at[c, pl.ds(s * RPT, RPT)])


def _deg_call(dst_d):
    fn = pl.kernel(
        _deg_body,
        out_type=jax.ShapeDtypeStruct((NC, NPAD), jnp.float32),
        mesh=_mesh(),
        compiler_params=pltpu.CompilerParams(use_tc_tiling_on_sc=False),
        scratch_types=[
            pltpu.VMEM((DCH, 128), jnp.int32),
            pltpu.VMEM((128,), jnp.float32),
            pltpu.VMEM((RPT,), jnp.float32),
            pltpu.VMEM_SHARED((NPAD,), jnp.float32),
        ],
    )
    return fn(dst_d)


def _prop_body(h_hbm, src_hbm, dst_hbm, out_hbm,
               src_v, dst_v, idx_v, dbuf, zbuf, acc, sem0, sem1):
    c = lax.axis_index("c")
    s = lax.axis_index("s")

    pltpu.sync_copy(src_hbm.at[c, s], src_v)
    pltpu.sync_copy(dst_hbm.at[c, s], dst_v)

    @pl.loop(0, zbuf.shape[0])
    def _(r):
        for v in range(FH // 16):
            zbuf[r, pl.ds(v * 16, 16)] = jnp.zeros((16,), jnp.float32)

    for k in range(K):
        for fh in range(NH):
            off = (k * NH + fh) * NPAD

            @pl.loop(0, CH)
            def _(r):
                for v in range(CW // 16):
                    sl = pl.ds(v * 16, 16)
                    idx_v[r, sl] = src_v[r, sl] + off

            for z in range(RPT // zbuf.shape[0]):
                pltpu.sync_copy(
                    zbuf, acc.at[pl.ds(s * RPT + z * zbuf.shape[0],
                                       zbuf.shape[0])])
            plsc.subcore_barrier()

            pltpu.async_copy(h_hbm.at[idx_v.at[0]], dbuf.at[0], sem0)

            @pl.loop(0, CH - 2, step=2)
            def _(j):
                pltpu.async_copy(h_hbm.at[idx_v.at[j + 1]], dbuf.at[1], sem1)
                pltpu.make_async_copy(h_hbm.at[idx_v.at[j]], dbuf.at[0], sem0).wait()
                pltpu.sync_copy(dbuf.at[0], acc.at[dst_v.at[j]], add=True)
                pltpu.async_copy(h_hbm.at[idx_v.at[j + 2]], dbuf.at[0], sem0)
                pltpu.make_async_copy(h_hbm.at[idx_v.at[j + 1]], dbuf.at[1], sem1).wait()
                pltpu.sync_copy(dbuf.at[1], acc.at[dst_v.at[j + 1]], add=True)

            pltpu.async_copy(h_hbm.at[idx_v.at[CH - 1]], dbuf.at[1], sem1)
            pltpu.make_async_copy(h_hbm.at[idx_v.at[CH - 2]], dbuf.at[0], sem0).wait()
            pltpu.sync_copy(dbuf.at[0], acc.at[dst_v.at[CH - 2]], add=True)
            pltpu.make_async_copy(h_hbm.at[idx_v.at[CH - 1]], dbuf.at[1], sem1).wait()
            pltpu.sync_copy(dbuf.at[1], acc.at[dst_v.at[CH - 1]], add=True)

            plsc.subcore_barrier()
            pltpu.sync_copy(acc.at[pl.ds(s * RPT, RPT)],
                            out_hbm.at[c, k, fh, pl.ds(s * RPT, RPT)])


def _prop_call(hs_flat, src_p, dst_p):
    fn = pl.kernel(
        _prop_body,
        out_type=jax.ShapeDtypeStruct((NC, K, NH, NPAD, FH), jnp.float32),
        mesh=_mesh(),
        compiler_params=pltpu.CompilerParams(use_tc_tiling_on_sc=False),
        scratch_types=[
            pltpu.VMEM((CH, CW), jnp.int32),
            pltpu.VMEM((CH, CW), jnp.int32),
            pltpu.VMEM((CH, CW), jnp.int32),
            pltpu.VMEM((2, CW, FH), jnp.float32),
            pltpu.VMEM((32, FH), jnp.float32),
            pltpu.VMEM_SHARED((NPAD, FH), jnp.float32),
            pltpu.SemaphoreType.DMA,
            pltpu.SemaphoreType.DMA,
        ],
    )
    return fn(hs_flat, src_p, dst_p)


def _psum(p_ref, k):
    return jnp.concatenate(
        [p_ref[0, k, fh] + p_ref[1, k, fh] for fh in range(NH)], axis=-1)


def _tc0_body(x_ref, deg_ref, iw_ref, rw_ref, b_ref, hs_ref, r_ref, dis_ref):
    xb = x_ref[...]
    deg = deg_ref[0] + deg_ref[1]
    dis = jnp.where(deg > 0, lax.rsqrt(deg), 0.0)
    dis_ref[...] = dis
    for k in range(K):
        h = jnp.dot(xb, iw_ref[k], preferred_element_type=jnp.float32) * dis
        for fh in range(NH):
            hs_ref[NH * k + fh] = h[:, fh * FH:(fh + 1) * FH]
    for t in range(T):
        for k in range(K):
            r_ref[t, k] = (jnp.dot(xb, rw_ref[t, k],
                                   preferred_element_type=jnp.float32)
                           + b_ref[t, k])


def _tc0_call(xpad, deg2c, init_w, root_w, b):
    grid = (NPAD // BN,)
    return pl.pallas_call(
        _tc0_body,
        grid=grid,
        in_specs=[
            pl.BlockSpec((BN, F), lambda i: (i, 0)),
            pl.BlockSpec((NC, BN, 1), lambda i: (0, i, 0)),
            pl.BlockSpec((K, F, F), lambda i: (0, 0, 0)),
            pl.BlockSpec((T, K, F, F), lambda i: (0, 0, 0, 0)),
            pl.BlockSpec((T, K, 1, F), lambda i: (0, 0, 0, 0)),
        ],
        out_specs=[
            pl.BlockSpec((K * NH, BN, FH), lambda i: (0, i, 0)),
            pl.BlockSpec((T, K, BN, F), lambda i: (0, 0, i, 0)),
            pl.BlockSpec((BN, 1), lambda i: (i, 0)),
        ],
        out_shape=[
            jax.ShapeDtypeStruct((K * NH, NPAD, FH), jnp.float32),
            jax.ShapeDtypeStruct((T, K, NPAD, F), jnp.float32),
            jax.ShapeDtypeStruct((NPAD, 1), jnp.float32),
        ],
    )(xpad, deg2c, init_w, root_w, b)


def _tc1_body(p_ref, r_ref, dis_ref, w_ref, hs_ref):
    dis = dis_ref[...]
    for k in range(K):
        o = jnp.maximum(_psum(p_ref, k) * dis + r_ref[0, k], 0.0)
        h = jnp.dot(o, w_ref[0, k], preferred_element_type=jnp.float32) * dis
        for fh in range(NH):
            hs_ref[NH * k + fh] = h[:, fh * FH:(fh + 1) * FH]


def _tc1_call(p, r, dis, w):
    grid = (NPAD // BN,)
    return pl.pallas_call(
        _tc1_body,
        grid=grid,
        in_specs=[
            pl.BlockSpec((NC, K, NH, BN, FH), lambda i: (0, 0, 0, i, 0)),
            pl.BlockSpec((T, K, BN, F), lambda i: (0, 0, i, 0)),
            pl.BlockSpec((BN, 1), lambda i: (i, 0)),
            pl.BlockSpec((T - 1, K, F, F), lambda i: (0, 0, 0, 0)),
        ],
        out_specs=[
            pl.BlockSpec((K * NH, BN, FH), lambda i: (0, i, 0)),
        ],
        out_shape=[
            jax.ShapeDtypeStruct((K * NH, NPAD, FH), jnp.float32),
        ],
    )(p, r, dis, w)[0]


def _tc2_body(p_ref, r_ref, dis_ref, iw_ref, rw_ref, b_ref, hs_ref, rn_ref):
    dis = dis_ref[...]
    outs = []
    for k in range(K):
        outs.append(jnp.maximum(_psum(p_ref, k) * dis + r_ref[1, k], 0.0))
    xs = 0.5 * (outs[0] + outs[1])
    for k in range(K):
        h = jnp.dot(xs, iw_ref[k], preferred_element_type=jnp.float32) * dis
        for fh in range(NH):
            hs_ref[NH * k + fh] = h[:, fh * FH:(fh + 1) * FH]
    for t in range(T):
        for k in range(K):
            rn_ref[t, k] = (jnp.dot(xs, rw_ref[t, k],
                                    preferred_element_type=jnp.float32)
                            + b_ref[t, k])


def _tc2_call(p, r, dis, init_w, root_w, b):
    grid = (NPAD // BN,)
    return pl.pallas_call(
        _tc2_body,
        grid=grid,
        in_specs=[
            pl.BlockSpec((NC, K, NH, BN, FH), lambda i: (0, 0, 0, i, 0)),
            pl.BlockSpec((T, K, BN, F), lambda i: (0, 0, i, 0)),
            pl.BlockSpec((BN, 1), lambda i: (i, 0)),
            pl.BlockSpec((K, F, F), lambda i: (0, 0, 0)),
            pl.BlockSpec((T, K, F, F), lambda i: (0, 0, 0, 0)),
            pl.BlockSpec((T, K, 1, F), lambda i: (0, 0, 0, 0)),
        ],
        out_specs=[
            pl.BlockSpec((K * NH, BN, FH), lambda i: (0, i, 0)),
            pl.BlockSpec((T, K, BN, F), lambda i: (0, 0, i, 0)),
        ],
        out_shape=[
            jax.ShapeDtypeStruct((K * NH, NPAD, FH), jnp.float32),
            jax.ShapeDtypeStruct((T, K, NPAD, F), jnp.float32),
        ],
    )(p, r, dis, init_w, root_w, b)


def _tcf_body(p_ref, r_ref, dis_ref, out_ref):
    dis = dis_ref[...]
    acc = None
    for k in range(K):
        o = jnp.maximum(_psum(p_ref, k) * dis + r_ref[1, k], 0.0)
        acc = o if acc is None else acc + o
    out_ref[...] = 0.5 * acc


def _tcf_call(p, r, dis):
    grid = (NPAD // BN,)
    return pl.pallas_call(
        _tcf_body,
        grid=grid,
        in_specs=[
            pl.BlockSpec((NC, K, NH, BN, FH), lambda i: (0, 0, 0, i, 0)),
            pl.BlockSpec((T, K, BN, F), lambda i: (0, 0, i, 0)),
            pl.BlockSpec((BN, 1), lambda i: (i, 0)),
        ],
        out_specs=[pl.BlockSpec((BN, F), lambda i: (i, 0))],
        out_shape=[jax.ShapeDtypeStruct((NPAD, F), jnp.float32)],
    )(p, r, dis)[0]


def kernel(x, edge_index, init_w0, w0, root_w0, b0, init_w1, w1, root_w1, b1,
           init_w2, w2, root_w2, b2):
    src = edge_index[0]
    dst = edge_index[1]
    pad = N + jnp.arange(EPAD - E, dtype=jnp.int32) % (NPAD - N)
    src_p = jnp.concatenate([src, pad]).reshape(NC, NS, CH, CW)
    dst_p = jnp.concatenate([dst, pad]).reshape(NC, NS, CH, CW)
    dst_d = jnp.concatenate([dst, pad]).reshape(NC * NS, DCH, 128)
    xpad = jnp.pad(x, ((0, NPAD - N), (0, 0)))

    deg2 = _deg_call(dst_d)
    deg2c = deg2[:, :, None]

    layers = [(init_w0, w0, root_w0, b0),
              (init_w1, w1, root_w1, b1),
              (init_w2, w2, root_w2, b2)]

    hs, r, dis = _tc0_call(xpad, deg2c, layers[0][0], layers[0][2], layers[0][3])
    for li in range(3):
        p0 = _prop_call(hs.reshape(K * NH * NPAD, FH), src_p, dst_p)
        hs1 = _tc1_call(p0, r, dis, layers[li][1])
        p1 = _prop_call(hs1.reshape(K * NH * NPAD, FH), src_p, dst_p)
        if li < 2:
            hs, r = _tc2_call(p1, r, dis, layers[li + 1][0],
                              layers[li + 1][2], layers[li + 1][3])
        else:
            out = _tcf_call(p1, r, dis)
    return out[:N]


# speedup vs baseline: 1.8294x; 1.0679x over previous
import functools

import jax
import jax.numpy as jnp
from jax import lax
from jax.experimental import pallas as pl
from jax.experimental.pallas import tpu as pltpu
from jax.experimental.pallas import tpu_sc as plsc

N = 10000
NPAD = 10240
E = 160000
F = 256
FH = 128
NH = 2
K = 2
T = 2
NC = 2
NS = 16
CW = 128
CH = 40
EPAD = NC * NS * CH * CW
DCH = EPAD // (NC * NS * 128)
RPT = NPAD // NS
BN = 256


def _mesh():
    return plsc.VectorSubcoreMesh(core_axis_name="c", subcore_axis_name="s")


def _deg_body(dst_hbm, out_hbm, dstv, onesv, zv, acc):
    c = lax.axis_index("c")
    s = lax.axis_index("s")
    w = s * NC + c

    for v8 in range(8):
        onesv[pl.ds(v8 * 16, 16)] = jnp.ones((16,), jnp.float32)

    @pl.loop(0, RPT // 16)
    def _(r):
        zv[pl.ds(r * 16, 16)] = jnp.zeros((16,), jnp.float32)

    pltpu.sync_copy(dst_hbm.at[w], dstv)
    pltpu.sync_copy(zv, acc.at[pl.ds(s * RPT, RPT)])
    plsc.subcore_barrier()

    @pl.loop(0, DCH)
    def _(j):
        pltpu.sync_copy(onesv, acc.at[dstv.at[j]], add=True)

    plsc.subcore_barrier()
    pltpu.sync_copy(acc.at[pl.ds(s * RPT, RPT)], out_hbm.at[c, pl.ds(s * RPT, RPT)])


def _deg_call(dst_d):
    fn = pl.kernel(
        _deg_body,
        out_type=jax.ShapeDtypeStruct((NC, NPAD), jnp.float32),
        mesh=_mesh(),
        compiler_params=pltpu.CompilerParams(use_tc_tiling_on_sc=False),
        scratch_types=[
            pltpu.VMEM((DCH, 128), jnp.int32),
            pltpu.VMEM((128,), jnp.float32),
            pltpu.VMEM((RPT,), jnp.float32),
            pltpu.VMEM_SHARED((NPAD,), jnp.float32),
        ],
    )
    return fn(dst_d)


def _prop_body(h_hbm, src_hbm, dst_hbm, out_hbm,
               src_v, dst_v, dbuf, zbuf, acc, sem0, sem1):
    c = lax.axis_index("c")
    s = lax.axis_index("s")

    pltpu.sync_copy(src_hbm.at[c, s], src_v)
    pltpu.sync_copy(dst_hbm.at[c, s], dst_v)

    @pl.loop(0, zbuf.shape[0])
    def _(r):
        for v in range(FH // 16):
            zbuf[r, pl.ds(v * 16, 16)] = jnp.zeros((16,), jnp.float32)

    for k in range(K):
        for fh in range(NH):
            if k or fh:
                @pl.loop(0, CH)
                def _(r):
                    for v in range(CW // 16):
                        sl = pl.ds(v * 16, 16)
                        src_v[r, sl] = src_v[r, sl] + NPAD

            for z in range(RPT // zbuf.shape[0]):
                pltpu.sync_copy(
                    zbuf, acc.at[pl.ds(s * RPT + z * zbuf.shape[0],
                                       zbuf.shape[0])])
            plsc.subcore_barrier()

            pltpu.async_copy(h_hbm.at[src_v.at[0]], dbuf.at[0], sem0)

            @pl.loop(0, CH - 2, step=2)
            def _(j):
                pltpu.async_copy(h_hbm.at[src_v.at[j + 1]], dbuf.at[1], sem1)
                pltpu.make_async_copy(h_hbm.at[src_v.at[j]], dbuf.at[0], sem0).wait()
                pltpu.sync_copy(dbuf.at[0], acc.at[dst_v.at[j]], add=True)
                pltpu.async_copy(h_hbm.at[src_v.at[j + 2]], dbuf.at[0], sem0)
                pltpu.make_async_copy(h_hbm.at[src_v.at[j + 1]], dbuf.at[1], sem1).wait()
                pltpu.sync_copy(dbuf.at[1], acc.at[dst_v.at[j + 1]], add=True)

            pltpu.async_copy(h_hbm.at[src_v.at[CH - 1]], dbuf.at[1], sem1)
            pltpu.make_async_copy(h_hbm.at[src_v.at[CH - 2]], dbuf.at[0], sem0).wait()
            pltpu.sync_copy(dbuf.at[0], acc.at[dst_v.at[CH - 2]], add=True)
            pltpu.make_async_copy(h_hbm.at[src_v.at[CH - 1]], dbuf.at[1], sem1).wait()
            pltpu.sync_copy(dbuf.at[1], acc.at[dst_v.at[CH - 1]], add=True)

            plsc.subcore_barrier()
            pltpu.sync_copy(acc.at[pl.ds(s * RPT, RPT)],
                            out_hbm.at[c, k, fh, pl.ds(s * RPT, RPT)])


def _prop_call(hs_flat, src_p, dst_p):
    fn = pl.kernel(
        _prop_body,
        out_type=jax.ShapeDtypeStruct((NC, K, NH, NPAD, FH), jnp.float32),
        mesh=_mesh(),
        compiler_params=pltpu.CompilerParams(use_tc_tiling_on_sc=False),
        scratch_types=[
            pltpu.VMEM((CH, CW), jnp.int32),
            pltpu.VMEM((CH, CW), jnp.int32),
            pltpu.VMEM((2, CW, FH), jnp.float32),
            pltpu.VMEM((16, FH), jnp.float32),
            pltpu.VMEM_SHARED((NPAD, FH), jnp.float32),
            pltpu.SemaphoreType.DMA,
            pltpu.SemaphoreType.DMA,
        ],
    )
    return fn(hs_flat, src_p, dst_p)


def _psum(p_ref, k):
    return jnp.concatenate(
        [p_ref[0, k, fh] + p_ref[1, k, fh] for fh in range(NH)], axis=-1)


def _tc0_body(x_ref, deg_ref, iw_ref, rw_ref, b_ref, hs_ref, r_ref, dis_ref):
    xb = x_ref[...]
    deg = deg_ref[0] + deg_ref[1]
    dis = jnp.where(deg > 0, lax.rsqrt(deg), 0.0)
    dis_ref[...] = dis
    for k in range(K):
        h = jnp.dot(xb, iw_ref[k], preferred_element_type=jnp.float32) * dis
        for fh in range(NH):
            hs_ref[NH * k + fh] = h[:, fh * FH:(fh + 1) * FH]
    for t in range(T):
        for k in range(K):
            r_ref[t, k] = (jnp.dot(xb, rw_ref[t, k],
                                   preferred_element_type=jnp.float32)
                           + b_ref[t, k])


def _tc0_call(xpad, deg2c, init_w, root_w, b):
    grid = (NPAD // BN,)
    return pl.pallas_call(
        _tc0_body,
        grid=grid,
        in_specs=[
            pl.BlockSpec((BN, F), lambda i: (i, 0)),
            pl.BlockSpec((NC, BN, 1), lambda i: (0, i, 0)),
            pl.BlockSpec((K, F, F), lambda i: (0, 0, 0)),
            pl.BlockSpec((T, K, F, F), lambda i: (0, 0, 0, 0)),
            pl.BlockSpec((T, K, 1, F), lambda i: (0, 0, 0, 0)),
        ],
        out_specs=[
            pl.BlockSpec((K * NH, BN, FH), lambda i: (0, i, 0)),
            pl.BlockSpec((T, K, BN, F), lambda i: (0, 0, i, 0)),
            pl.BlockSpec((BN, 1), lambda i: (i, 0)),
        ],
        out_shape=[
            jax.ShapeDtypeStruct((K * NH, NPAD, FH), jnp.float32),
            jax.ShapeDtypeStruct((T, K, NPAD, F), jnp.float32),
            jax.ShapeDtypeStruct((NPAD, 1), jnp.float32),
        ],
    )(xpad, deg2c, init_w, root_w, b)


def _tc1_body(p_ref, r_ref, dis_ref, w_ref, hs_ref):
    dis = dis_ref[...]
    for k in range(K):
        o = jnp.maximum(_psum(p_ref, k) * dis + r_ref[0, k], 0.0)
        h = jnp.dot(o, w_ref[0, k], preferred_element_type=jnp.float32) * dis
        for fh in range(NH):
            hs_ref[NH * k + fh] = h[:, fh * FH:(fh + 1) * FH]


def _tc1_call(p, r, dis, w):
    grid = (NPAD // BN,)
    return pl.pallas_call(
        _tc1_body,
        grid=grid,
        in_specs=[
            pl.BlockSpec((NC, K, NH, BN, FH), lambda i: (0, 0, 0, i, 0)),
            pl.BlockSpec((T, K, BN, F), lambda i: (0, 0, i, 0)),
            pl.BlockSpec((BN, 1), lambda i: (i, 0)),
            pl.BlockSpec((T - 1, K, F, F), lambda i: (0, 0, 0, 0)),
        ],
        out_specs=[
            pl.BlockSpec((K * NH, BN, FH), lambda i: (0, i, 0)),
        ],
        out_shape=[
            jax.ShapeDtypeStruct((K * NH, NPAD, FH), jnp.float32),
        ],
    )(p, r, dis, w)[0]


def _tc2_body(p_ref, r_ref, dis_ref, iw_ref, rw_ref, b_ref, hs_ref, rn_ref):
    dis = dis_ref[...]
    outs = []
    for k in range(K):
        outs.append(jnp.maximum(_psum(p_ref, k) * dis + r_ref[1, k], 0.0))
    xs = 0.5 * (outs[0] + outs[1])
    for k in range(K):
        h = jnp.dot(xs, iw_ref[k], preferred_element_type=jnp.float32) * dis
        for fh in range(NH):
            hs_ref[NH * k + fh] = h[:, fh * FH:(fh + 1) * FH]
    for t in range(T):
        for k in range(K):
            rn_ref[t, k] = (jnp.dot(xs, rw_ref[t, k],
                                    preferred_element_type=jnp.float32)
                            + b_ref[t, k])


def _tc2_call(p, r, dis, init_w, root_w, b):
    grid = (NPAD // BN,)
    return pl.pallas_call(
        _tc2_body,
        grid=grid,
        in_specs=[
            pl.BlockSpec((NC, K, NH, BN, FH), lambda i: (0, 0, 0, i, 0)),
            pl.BlockSpec((T, K, BN, F), lambda i: (0, 0, i, 0)),
            pl.BlockSpec((BN, 1), lambda i: (i, 0)),
            pl.BlockSpec((K, F, F), lambda i: (0, 0, 0)),
            pl.BlockSpec((T, K, F, F), lambda i: (0, 0, 0, 0)),
            pl.BlockSpec((T, K, 1, F), lambda i: (0, 0, 0, 0)),
        ],
        out_specs=[
            pl.BlockSpec((K * NH, BN, FH), lambda i: (0, i, 0)),
            pl.BlockSpec((T, K, BN, F), lambda i: (0, 0, i, 0)),
        ],
        out_shape=[
            jax.ShapeDtypeStruct((K * NH, NPAD, FH), jnp.float32),
            jax.ShapeDtypeStruct((T, K, NPAD, F), jnp.float32),
        ],
    )(p, r, dis, init_w, root_w, b)


def _tcf_body(p_ref, r_ref, dis_ref, out_ref):
    dis = dis_ref[...]
    acc = None
    for k in range(K):
        o = jnp.maximum(_psum(p_ref, k) * dis + r_ref[1, k], 0.0)
        acc = o if acc is None else acc + o
    out_ref[...] = 0.5 * acc


def _tcf_call(p, r, dis):
    grid = (NPAD // BN,)
    return pl.pallas_call(
        _tcf_body,
        grid=grid,
        in_specs=[
            pl.BlockSpec((NC, K, NH, BN, FH), lambda i: (0, 0, 0, i, 0)),
            pl.BlockSpec((T, K, BN, F), lambda i: (0, 0, i, 0)),
            pl.BlockSpec((BN, 1), lambda i: (i, 0)),
        ],
        out_specs=[pl.BlockSpec((BN, F), lambda i: (i, 0))],
        out_shape=[jax.ShapeDtypeStruct((NPAD, F), jnp.float32)],
    )(p, r, dis)[0]


def kernel(x, edge_index, init_w0, w0, root_w0, b0, init_w1, w1, root_w1, b1,
           init_w2, w2, root_w2, b2):
    src = edge_index[0]
    dst = edge_index[1]
    pad = N + jnp.arange(EPAD - E, dtype=jnp.int32) % (NPAD - N)
    src_p = jnp.concatenate([src, pad]).reshape(NC, NS, CH, CW)
    dst_p = jnp.concatenate([dst, pad]).reshape(NC, NS, CH, CW)
    dst_d = jnp.concatenate([dst, pad]).reshape(NC * NS, DCH, 128)
    xpad = jnp.pad(x, ((0, NPAD - N), (0, 0)))

    deg2 = _deg_call(dst_d)
    deg2c = deg2[:, :, None]

    layers = [(init_w0, w0, root_w0, b0),
              (init_w1, w1, root_w1, b1),
              (init_w2, w2, root_w2, b2)]

    hs, r, dis = _tc0_call(xpad, deg2c, layers[0][0], layers[0][2], layers[0][3])
    for li in range(3):
        p0 = _prop_call(hs.reshape(K * NH * NPAD, FH), src_p, dst_p)
        hs1 = _tc1_call(p0, r, dis, layers[li][1])
        p1 = _prop_call(hs1.reshape(K * NH * NPAD, FH), src_p, dst_p)
        if li < 2:
            hs, r = _tc2_call(p1, r, dis, layers[li + 1][0],
                              layers[li + 1][2], layers[li + 1][3])
        else:
            out = _tcf_call(p1, r, dis)
    return out[:N]
